# Initial kernel scaffold; baseline (speedup 1.0000x reference)
import jax
import jax.numpy as jnp
from jax.experimental import pallas as pl


def kernel(x, edge_index, W1, b1, W2, b2, W3, b3, g1, be1, g2, be2, g3, be3, Wf, bf):
    raise NotImplementedError("write your pallas kernel here")


# speedup vs baseline: 9.3158x; 9.3158x over previous
import functools

import jax
import jax.numpy as jnp
from jax import lax
from jax.experimental import pallas as pl
from jax.experimental.pallas import tpu as pltpu
from jax.experimental.pallas import tpu_sc as plsc

N = 10000
D = 128
C = 40
NP = 10112
RPT = NP // 16
NSC = 2
NTPC = 16
NTILES = NSC * NTPC
CH = 128
NCH = 80
EPT = CH * NCH
EP = EPT * NTILES

_f32 = jnp.float32


def _mesh():
    return plsc.VectorSubcoreMesh(
        core_axis_name="c", subcore_axis_name="s",
        num_cores=NSC, num_subcores=NTPC)


def _sc_degree(col_flat):
    @functools.partial(
        pl.kernel,
        out_type=jax.ShapeDtypeStruct((NTILES, NP), _f32),
        mesh=_mesh(),
        compiler_params=pltpu.CompilerParams(needs_layout_passes=False),
        scratch_types=[
            pltpu.VMEM((EPT,), jnp.int32),
            pltpu.VMEM((NP,), _f32),
        ],
    )
    def k(col_hbm, out_hbm, col_v, cnt_v):
        c = lax.axis_index("c")
        s = lax.axis_index("s")
        g = c * NTPC + s
        pltpu.sync_copy(col_hbm.at[g], col_v)
        zeros16 = jnp.zeros((16,), _f32)

        def zbody(i, carry):
            cnt_v[pl.ds(i * 16, 16)] = zeros16
            return carry
        lax.fori_loop(0, NP // 16, zbody, 0)

        ones16 = jnp.ones((16,), _f32)

        def hbody(i, carry):
            idx = col_v[pl.ds(i * 16, 16)]
            plsc.addupdate_scatter(cnt_v, [idx], ones16)
            return carry
        lax.fori_loop(0, EPT // 16, hbody, 0)
        pltpu.sync_copy(cnt_v, out_hbm.at[g])

    return k(col_flat)


FAST_C = 0
NCH_FAST = 112
NCH_SLOW = 48
TOTCH = NTPC * (NCH_FAST + NCH_SLOW)


def _sc_scatter(hs, idx3, zrows):
    @functools.partial(
        pl.kernel,
        out_type=jax.ShapeDtypeStruct((NSC, NP, D), _f32),
        mesh=_mesh(),
        compiler_params=pltpu.CompilerParams(needs_layout_passes=False),
        scratch_types=[
            pltpu.VMEM((NCH_FAST // 2, 2, CH), jnp.int32),
            pltpu.VMEM((2 * CH, D), _f32),
            pltpu.VMEM_SHARED((NP, D), _f32),
            [pltpu.SemaphoreType.DMA] * 4,
            pltpu.SemaphoreType.DMA,
        ],
    )
    def k(hs_hbm, idx_hbm, z_hbm, out_hbm, idx_v, buf, acc, sems, zsem):
        c = lax.axis_index("c")
        s = lax.axis_index("s")
        zcp = pltpu.async_copy(z_hbm, acc.at[pl.ds(s * RPT, RPT)], zsem)

        HB = CH // 2

        def gsub(j, half_sel, slot):
            return pltpu.async_copy(
                hs_hbm.at[idx_v.at[j, 1, pl.ds(half_sel * HB, HB)]],
                buf.at[pl.ds(slot * HB, HB)], sems[slot])

        def gwait(j, half_sel, slot):
            pltpu.make_async_copy(
                hs_hbm.at[idx_v.at[j, 1, pl.ds(half_sel * HB, HB)]],
                buf.at[pl.ds(slot * HB, HB)], sems[slot]).wait()

        def run(nch, base):
            half = nch // 2
            for h in (0, 1):
                pltpu.sync_copy(idx_hbm.at[pl.ds(base + h * half, half)],
                                idx_v.at[pl.ds(0, half)])
                gsub(0, 0, 0)
                gsub(0, 1, 1)
                if h == 0:
                    zcp.wait()
                    plsc.subcore_barrier()

                @pl.loop(0, half, step=2)
                def _(p0):
                    for pb in (0, 1):
                        p = p0 + pb
                        s0 = 2 * pb
                        n0 = 2 * (1 - pb)

                        @pl.when(p + 1 < half)
                        def _():
                            gsub(p + 1, 0, n0)
                            gsub(p + 1, 1, n0 + 1)

                        gwait(p, 0, s0)
                        gwait(p, 1, s0 + 1)
                        pltpu.sync_copy(buf.at[pl.ds(s0 * HB, CH)],
                                        acc.at[idx_v.at[p, 0]], add=True)

        @pl.when(c == FAST_C)
        def _():
            run(NCH_FAST, s * NCH_FAST)

        @pl.when(c != FAST_C)
        def _():
            run(NCH_SLOW, NTPC * NCH_FAST + s * NCH_SLOW)

        plsc.subcore_barrier()
        pltpu.sync_copy(acc.at[pl.ds(s * RPT, RPT)],
                        out_hbm.at[c, pl.ds(s * RPT, RPT)])

    return k(hs, idx3, zrows)


def _gelu(x):
    return 0.5 * x * (1.0 + lax.erf(x * (2.0 ** -0.5)))


def _tc_prep(cnt, xp, w1):
    def f(cnt_ref, x_ref, w_ref, dis_ref, hs_ref):
        total = jnp.sum(cnt_ref[...], axis=0, keepdims=True)
        disr = lax.rsqrt(total + 1.0)
        mask = lax.broadcasted_iota(jnp.int32, (1, NP), 1) < N
        dis = jnp.where(mask, disr, 0.0).reshape(NP, 1)
        dis_ref[...] = dis
        hs_ref[...] = jnp.dot(x_ref[...], w_ref[...],
                              preferred_element_type=_f32) * dis

    return pl.pallas_call(
        f,
        out_shape=[jax.ShapeDtypeStruct((NP, 1), _f32),
                   jax.ShapeDtypeStruct((NP, D), _f32)],
    )(cnt, xp, w1)


def _bn_gelu(acc_ref, hs_ref, dis_ref, b_ref, g_ref, be_ref):
    pre = (acc_ref[0] + acc_ref[1] + hs_ref[...]) * dis_ref[...] + b_ref[...]
    pre_n = pre[:N]
    mean = jnp.mean(pre_n, axis=0, keepdims=True)
    var = jnp.mean(jnp.square(pre_n - mean), axis=0, keepdims=True)
    bn = g_ref[...] * (pre - mean) * lax.rsqrt(var + 1e-5) + be_ref[...]
    return _gelu(bn)


def _tc_layer(acc, hs, dis, b, g, be, w_next):
    def f(acc_ref, hs_ref, dis_ref, b_ref, g_ref, be_ref, w_ref, out_ref):
        act = _bn_gelu(acc_ref, hs_ref, dis_ref, b_ref, g_ref, be_ref)
        out_ref[...] = jnp.dot(act, w_ref[...],
                               preferred_element_type=_f32) * dis_ref[...]

    return pl.pallas_call(
        f, out_shape=jax.ShapeDtypeStruct((NP, D), _f32),
    )(acc, hs, dis, b, g, be, w_next)


def _tc_final(acc, hs, dis, b, g, be, wf, bf):
    def f(acc_ref, hs_ref, dis_ref, b_ref, g_ref, be_ref, w_ref, bf_ref, out_ref):
        act = _bn_gelu(acc_ref, hs_ref, dis_ref, b_ref, g_ref, be_ref)
        out_ref[...] = jnp.dot(act, w_ref[...],
                               preferred_element_type=_f32) + bf_ref[...]

    return pl.pallas_call(
        f, out_shape=jax.ShapeDtypeStruct((NP, D), _f32),
    )(acc, hs, dis, b, g, be, wf, bf)


def kernel(x, edge_index, W1, b1, W2, b2, W3, b3,
           g1, be1, g2, be2, g3, be3, Wf, bf):
    E = edge_index.shape[1]
    row = edge_index[0]
    col = edge_index[1]
    pad = jnp.full((EP - E,), N, jnp.int32)
    pad_col = (N + 1) + (jnp.arange(EP - E, dtype=jnp.int32) % (NP - N - 1))
    rowp = jnp.concatenate([row, pad])
    colp = jnp.concatenate([col, pad_col])
    idx3 = jnp.stack([colp.reshape(TOTCH, CH), rowp.reshape(TOTCH, CH)], axis=1)
    col_flat = colp.reshape(NTILES, EPT)
    xp = jnp.concatenate([x, jnp.zeros((NP - N, D), _f32)], axis=0)
    zrows = jnp.zeros((RPT, D), _f32)
    b1r, b2r, b3r = b1.reshape(1, D), b2.reshape(1, D), b3.reshape(1, D)
    g1r, g2r, g3r = g1.reshape(1, D), g2.reshape(1, D), g3.reshape(1, D)
    be1r, be2r, be3r = be1.reshape(1, D), be2.reshape(1, D), be3.reshape(1, D)
    wf_p = jnp.pad(Wf, ((0, 0), (0, D - C)))
    bf_p = jnp.pad(bf, (0, D - C)).reshape(1, D)

    cnt = _sc_degree(col_flat)
    dis, hs1 = _tc_prep(cnt, xp, W1)
    acc1 = _sc_scatter(hs1, idx3, zrows)
    hs2 = _tc_layer(acc1, hs1, dis, b1r, g1r, be1r, W2)
    acc2 = _sc_scatter(hs2, idx3, zrows)
    hs3 = _tc_layer(acc2, hs2, dis, b2r, g2r, be2r, W3)
    acc3 = _sc_scatter(hs3, idx3, zrows)
    outp = _tc_final(acc3, hs3, dis, b3r, g3r, be3r, wf_p, bf_p)
    return outp[:N, :C]
